# baseline (speedup 1.0000x reference)
import functools

import jax
import jax.numpy as jnp
from jax import lax
from jax.experimental import pallas as pl
from jax.experimental.pallas import tpu as pltpu
from jax.experimental.pallas import tpu_sc as plsc

DIM = 64
BATCH = 16384
HIST = 200
TOTAL = BATCH * HIST
SUB = 128
ROWS = TOTAL // SUB
NC = 2
NS = 16
NW = NC * NS
NSUB = 5
CHUNK = NSUB * SUB
PER_W = TOTAL // NW
NCHUNK = PER_W // CHUNK


def _make_sc_gather():
  mesh = plsc.VectorSubcoreMesh(core_axis_name="c", subcore_axis_name="s")

  @functools.partial(
      pl.kernel,
      mesh=mesh,
      out_type=jax.ShapeDtypeStruct((ROWS, SUB, DIM), jnp.float32),
      compiler_params=pltpu.CompilerParams(use_tc_tiling_on_sc=False),
      scratch_types=[
          pltpu.VMEM((CHUNK,), jnp.int32),
          pltpu.VMEM((NSUB, SUB, DIM), jnp.float32),
          pltpu.SemaphoreType.DMA,
      ],
  )
  def sc_gather(x_hbm, table_hbm, out_hbm, idx_v, rows_v, sem):
    wid = lax.axis_index("s") * NC + lax.axis_index("c")
    base = wid * PER_W
    row0 = wid * (PER_W // SUB)

    def body(g, carry):
      pltpu.sync_copy(x_hbm.at[pl.ds(base + g * CHUNK, CHUNK)], idx_v)
      handles = [
          pltpu.async_copy(
              table_hbm.at[idx_v.at[pl.ds(j * SUB, SUB)]], rows_v.at[j], sem)
          for j in range(NSUB)
      ]
      for h in handles:
        h.wait()
      pltpu.sync_copy(rows_v, out_hbm.at[pl.ds(row0 + g * NSUB, NSUB)])
      return carry

    lax.fori_loop(0, NCHUNK, body, 0)

  return sc_gather


_sc_gather = _make_sc_gather()


@jax.jit
def kernel(x, table):
  xf = x.reshape(TOTAL).astype(jnp.int32)
  out = _sc_gather(xf, table)
  return out.reshape(BATCH, HIST, DIM)


# speedup vs baseline: 1.0606x; 1.0606x over previous
import functools

import jax
import jax.numpy as jnp
from jax import lax
from jax.experimental import pallas as pl
from jax.experimental.pallas import tpu as pltpu
from jax.experimental.pallas import tpu_sc as plsc

DIM = 64
BATCH = 16384
HIST = 200
TOTAL = BATCH * HIST
SUB = 128
ROWS = TOTAL // SUB
NC = 2
NS = 16
NW = NC * NS
NSUB = 5
CHUNK = NSUB * SUB
PER_W = TOTAL // NW
NCHUNK = PER_W // CHUNK


def _make_sc_gather():
  mesh = plsc.VectorSubcoreMesh(core_axis_name="c", subcore_axis_name="s")

  @functools.partial(
      pl.kernel,
      mesh=mesh,
      out_type=jax.ShapeDtypeStruct((ROWS, SUB, DIM), jnp.float32),
      compiler_params=pltpu.CompilerParams(use_tc_tiling_on_sc=False),
      scratch_types=[
          pltpu.VMEM((2, CHUNK), jnp.int32),
          pltpu.VMEM((2, NSUB, SUB, DIM), jnp.float32),
          pltpu.SemaphoreType.DMA,
          pltpu.SemaphoreType.DMA,
          pltpu.SemaphoreType.DMA,
          pltpu.SemaphoreType.DMA,
          pltpu.SemaphoreType.DMA,
          pltpu.SemaphoreType.DMA,
      ],
  )
  def sc_gather(x_hbm, table_hbm, out_hbm, idx_v, rows_v,
                is0, is1, gs0, gs1, ss0, ss1):
    wid = lax.axis_index("s") * NC + lax.axis_index("c")
    base = wid * PER_W
    row0 = wid * (PER_W // SUB)
    isem = (is0, is1)
    gsem = (gs0, gs1)
    ssem = (ss0, ss1)

    def load_idx(c, b):
      pltpu.async_copy(x_hbm.at[pl.ds(base + c * CHUNK, CHUNK)],
                       idx_v.at[b], isem[b])

    def wait_idx(b):
      pltpu.make_async_copy(x_hbm.at[pl.ds(0, CHUNK)], idx_v.at[b],
                            isem[b]).wait()

    def gather(b):
      handles = [
          pltpu.async_copy(
              table_hbm.at[idx_v.at[b, pl.ds(j * SUB, SUB)]],
              rows_v.at[b, j], gsem[b])
          for j in range(NSUB)
      ]
      for h in handles:
        h.wait()

    def store(c, b):
      pltpu.async_copy(rows_v.at[b], out_hbm.at[pl.ds(row0 + c * NSUB, NSUB)],
                       ssem[b])

    def wait_store(b):
      pltpu.make_async_copy(rows_v.at[b], out_hbm.at[pl.ds(0, NSUB)],
                            ssem[b]).wait()

    load_idx(0, 0)
    load_idx(1, 1)
    for b in range(2):
      wait_idx(b)
      gather(b)
      store(b, b)
      load_idx(b + 2, b)

    def body(g, carry):
      for b in range(2):
        c = 2 + g * 2 + b
        wait_idx(b)
        wait_store(b)
        gather(b)
        store(c, b)
        load_idx(jnp.minimum(c + 2, NCHUNK - 1), b)
      return carry

    lax.fori_loop(0, (NCHUNK - 2) // 2, body, 0, unroll=False)

    for b in range(2):
      wait_store(b)
      wait_idx(b)

  return sc_gather


_sc_gather = _make_sc_gather()


@jax.jit
def kernel(x, table):
  xf = x.reshape(TOTAL).astype(jnp.int32)
  out = _sc_gather(xf, table)
  return out.reshape(BATCH, HIST, DIM)
